# baseline (speedup 1.0000x reference)
import jax
import jax.numpy as jnp
from jax import lax
from jax.experimental import pallas as pl
from jax.experimental.pallas import tpu as pltpu
from jax.experimental.pallas import tpu_sc as plsc

_L, _B, _D = 2048, 2, 1024
_E, _K, _F = 8, 2, 4096
_N = _L * _B
_EP = 128
_TM = 512
_NT = _N // _TM
_NEG = -1e30

_TM2 = 256
_NTILE = _N * _K // _TM2 + _E
_NSLOT = _NTILE * _TM2

_NC, _NSUB = 2, 16
_NW = _NC * _NSUB
_TPW = _N // _NW
_CH = 32
_NCH = _TPW // _CH
_CH2 = 16
_NCH2 = _TPW // _CH2


def _gate_kernel(x_ref, wgt_ref, wgi_ref, bgt_ref, bgi_ref, tp_ref, wt_ref,
                 bt_ref, alpha_ref, it_ref,
                 mi_ref, mf_ref, stats_ref, cntc_ref, laux_ref, basel_ref,
                 te_ref):
    t = pl.program_id(0)
    a = alpha_ref[0, 0]
    use_txt = it_ref[0, 0] != 0
    wg = jnp.where(use_txt, wgt_ref[...], wgi_ref[...])
    bg = jnp.where(use_txt, bgt_ref[...], bgi_ref[...])
    x = x_ref[...]
    il = lax.dot_general(x, wg, (((1,), (0,)), ((), ())),
                         preferred_element_type=jnp.float32) + bg
    tl = lax.dot_general(tp_ref[...], wt_ref[...], (((1,), (0,)), ((), ())),
                         preferred_element_type=jnp.float32) + bt_ref[...]
    g = (1.0 - a) * il + a * tl
    lane = lax.broadcasted_iota(jnp.int32, (_TM, _EP), 1)
    valid = lane < _E
    g = jnp.where(valid, g, _NEG)
    m1 = jnp.max(g, axis=1, keepdims=True)
    i1 = jnp.min(jnp.where(g == m1, lane, _EP), axis=1, keepdims=True)
    sel1 = lane == i1
    g2 = jnp.where(sel1, _NEG, g)
    m2 = jnp.max(g2, axis=1, keepdims=True)
    i2 = jnp.min(jnp.where(g2 == m2, lane, _EP), axis=1, keepdims=True)
    sel2 = lane == i2
    d = jnp.exp(m2 - m1)
    w1 = 1.0 / (1.0 + d)
    w2 = d / (1.0 + d)
    ez = jnp.where(valid, jnp.exp(g - m1), 0.0)
    p = ez / jnp.sum(ez, axis=1, keepdims=True)
    sump = jnp.sum(p, axis=0, keepdims=True)
    m = jnp.where(sel1 | sel2, 1.0, 0.0)
    cnt = jnp.sum(m, axis=0, keepdims=True)

    @pl.when(t == 0)
    def _():
        stats_ref[...] = jnp.zeros_like(stats_ref)
        cntc_ref[...] = jnp.zeros_like(cntc_ref)

    @pl.when(t < _NT)
    def _():
        carry = stats_ref[1:2, :]
        row = lax.broadcasted_iota(jnp.int32, (_TM, _TM), 0)
        col = lax.broadcasted_iota(jnp.int32, (_TM, _TM), 1)
        ltri = jnp.where(col < row, 1.0, 0.0).astype(jnp.bfloat16)
        excl = lax.dot_general(ltri, m.astype(jnp.bfloat16),
                               (((1,), (0,)), ((), ())),
                               preferred_element_type=jnp.float32)
        rank = carry + excl
        r1 = jnp.sum(jnp.where(sel1, rank, 0.0), axis=1, keepdims=True)
        r2 = jnp.sum(jnp.where(sel2, rank, 0.0), axis=1, keepdims=True)
        lane8 = lax.broadcasted_iota(jnp.int32, (_TM, 8), 1)
        mi = (jnp.where(lane8 == 0, i1, 0) + jnp.where(lane8 == 1, i2, 0)
              + jnp.where(lane8 == 2, r1.astype(jnp.int32), 0)
              + jnp.where(lane8 == 3, r2.astype(jnp.int32), 0))
        mf = jnp.where(lane8 == 0, w1, 0.0) + jnp.where(lane8 == 1, w2, 0.0)
        mi_ref[...] = mi
        mf_ref[...] = mf
        stats_ref[...] += jnp.concatenate([sump, cnt], axis=0)
        cntc_ref[...] += lax.dot_general(
            m.astype(jnp.bfloat16), jnp.ones((_TM, _EP), jnp.bfloat16),
            (((0,), (0,)), ((), ())), preferred_element_type=jnp.float32)

    @pl.when(t == _NT)
    def _():
        s = stats_ref[...]
        lx = jnp.sum(s[0:1, :] * s[1:2, :]) * (1.0 / (_N * _N))
        laux_ref[...] = jnp.full((1, _EP), lx, jnp.float32)
        sub = lax.broadcasted_iota(jnp.int32, (_EP, _EP), 0)
        lan = lax.broadcasted_iota(jnp.int32, (_EP, _EP), 1)
        cc = jnp.where(sub < _E, cntc_ref[...], 0.0)
        cpad = jnp.ceil(cc * (1.0 / _TM2)) * float(_TM2)
        ustrict = jnp.where(sub < lan, 1.0, 0.0).astype(jnp.bfloat16)
        bases = lax.dot_general(ustrict, cpad.astype(jnp.bfloat16),
                                (((0,), (0,)), ((), ())),
                                preferred_element_type=jnp.float32)
        basel = jnp.sum(jnp.where(sub == lan, bases, 0.0), axis=0,
                        keepdims=True)
        basel_ref[...] = basel.astype(jnp.int32)
        btiles = bases * (1.0 / _TM2)
        cmp = jnp.where((sub < _E) & (btiles <= lan.astype(jnp.float32)),
                        1.0, 0.0)
        te_ref[...] = (jnp.sum(cmp, axis=0, keepdims=True)
                       - 1.0).astype(jnp.int32)


def _run_gate(x2d, wgt, wgi, bgt, bgi, tp, wt, btp, a, it):
    last = _NT - 1
    return pl.pallas_call(
        _gate_kernel,
        grid=(_NT + 1,),
        in_specs=[
            pl.BlockSpec((_TM, _D), lambda t: (jnp.minimum(t, last), 0)),
            pl.BlockSpec((_D, _EP), lambda t: (0, 0)),
            pl.BlockSpec((_D, _EP), lambda t: (0, 0)),
            pl.BlockSpec((1, _EP), lambda t: (0, 0)),
            pl.BlockSpec((1, _EP), lambda t: (0, 0)),
            pl.BlockSpec((1, _D), lambda t: (0, 0)),
            pl.BlockSpec((_D, _EP), lambda t: (0, 0)),
            pl.BlockSpec((1, _EP), lambda t: (0, 0)),
            pl.BlockSpec((1, 1), lambda t: (0, 0)),
            pl.BlockSpec((1, 1), lambda t: (0, 0)),
        ],
        out_specs=[
            pl.BlockSpec((_TM, 8), lambda t: (jnp.minimum(t, last), 0)),
            pl.BlockSpec((_TM, 8), lambda t: (jnp.minimum(t, last), 0)),
            pl.BlockSpec((2, _EP), lambda t: (0, 0)),
            pl.BlockSpec((_EP, _EP), lambda t: (0, 0)),
            pl.BlockSpec((1, _EP), lambda t: (0, 0)),
            pl.BlockSpec((1, _EP), lambda t: (0, 0)),
            pl.BlockSpec((1, _EP), lambda t: (0, 0)),
        ],
        out_shape=[
            jax.ShapeDtypeStruct((_N, 8), jnp.int32),
            jax.ShapeDtypeStruct((_N, 8), jnp.float32),
            jax.ShapeDtypeStruct((2, _EP), jnp.float32),
            jax.ShapeDtypeStruct((_EP, _EP), jnp.float32),
            jax.ShapeDtypeStruct((1, _EP), jnp.float32),
            jax.ShapeDtypeStruct((1, _EP), jnp.int32),
            jax.ShapeDtypeStruct((1, _EP), jnp.int32),
        ],
    )(x2d, wgt, wgi, bgt, bgi, tp, wt, btp, a, it)


def _pos_kernel(mi_ref, mf_ref, basel_ref, x_ref,
                p1_ref, p2_ref, w1_ref, w2_ref, xb_ref):
    xb_ref[...] = x_ref[...].astype(jnp.bfloat16)
    mi = mi_ref[...]
    mf = mf_ref[...]
    lane8 = lax.broadcasted_iota(jnp.int32, (_TM, 8), 1)
    e1 = jnp.sum(jnp.where(lane8 == 0, mi, 0), axis=1, keepdims=True)
    e2 = jnp.sum(jnp.where(lane8 == 1, mi, 0), axis=1, keepdims=True)
    r1 = jnp.sum(jnp.where(lane8 == 2, mi, 0), axis=1, keepdims=True)
    r2 = jnp.sum(jnp.where(lane8 == 3, mi, 0), axis=1, keepdims=True)
    wv1 = jnp.sum(jnp.where(lane8 == 0, mf, 0.0), axis=1, keepdims=True)
    wv2 = jnp.sum(jnp.where(lane8 == 1, mf, 0.0), axis=1, keepdims=True)
    lane = lax.broadcasted_iota(jnp.int32, (_TM, _EP), 1)
    b = basel_ref[...]
    b1 = jnp.sum(jnp.where(lane == e1, b, 0), axis=1, keepdims=True)
    b2 = jnp.sum(jnp.where(lane == e2, b, 0), axis=1, keepdims=True)
    p1_ref[...] = b1 + r1
    p2_ref[...] = b2 + r2
    w1_ref[...] = jnp.broadcast_to(wv1, (_TM, _EP))
    w2_ref[...] = jnp.broadcast_to(wv2, (_TM, _EP))


def _run_pos(mi, mf, basel, x2d):
    return pl.pallas_call(
        _pos_kernel,
        grid=(_NT,),
        in_specs=[
            pl.BlockSpec((_TM, 8), lambda t: (t, 0)),
            pl.BlockSpec((_TM, 8), lambda t: (t, 0)),
            pl.BlockSpec((1, _EP), lambda t: (0, 0)),
            pl.BlockSpec((_TM, _D), lambda t: (t, 0)),
        ],
        out_specs=[
            pl.BlockSpec((_TM, 1), lambda t: (t, 0)),
            pl.BlockSpec((_TM, 1), lambda t: (t, 0)),
            pl.BlockSpec((_TM, _EP), lambda t: (t, 0)),
            pl.BlockSpec((_TM, _EP), lambda t: (t, 0)),
            pl.BlockSpec((_TM, _D), lambda t: (t, 0)),
        ],
        out_shape=[
            jax.ShapeDtypeStruct((_N, 1), jnp.int32),
            jax.ShapeDtypeStruct((_N, 1), jnp.int32),
            jax.ShapeDtypeStruct((_N, _EP), jnp.float32),
            jax.ShapeDtypeStruct((_N, _EP), jnp.float32),
            jax.ShapeDtypeStruct((_N, _D), jnp.bfloat16),
        ],
    )(mi, mf, basel, x2d)


def _dispatch_kernel(x_hbm, p1_hbm, p2_hbm, xs_hbm,
                     pb1, pb2, xbuf, sem):
    wid = lax.axis_index("s") * _NC + lax.axis_index("c")
    tb = wid * _TPW
    pltpu.sync_copy(p1_hbm.at[pl.ds(wid * _NCH, _NCH)], pb1)
    pltpu.sync_copy(p2_hbm.at[pl.ds(wid * _NCH, _NCH)], pb2)
    cps = []
    for c in range(_NCH):
        b = c % 2
        if c >= 2:
            for cp in cps[2 * (c - 2):2 * (c - 1)]:
                cp.wait()
        pltpu.sync_copy(x_hbm.at[pl.ds(tb + _CH * c, _CH)], xbuf.at[b])
        cps.append(pltpu.async_copy(xbuf.at[b], xs_hbm.at[pb1.at[c]], sem))
        cps.append(pltpu.async_copy(xbuf.at[b], xs_hbm.at[pb2.at[c]], sem))
    for cp in cps[2 * (_NCH - 2):]:
        cp.wait()


def _run_dispatch(x2d, p1r, p2r):
    mesh = plsc.VectorSubcoreMesh(core_axis_name="c", subcore_axis_name="s")
    f = pl.kernel(
        _dispatch_kernel,
        mesh=mesh,
        out_type=jax.ShapeDtypeStruct((_NSLOT, _D // 2), jnp.int32),
        scratch_types=[
            pltpu.VMEM((_NCH, _CH), jnp.int32),
            pltpu.VMEM((_NCH, _CH), jnp.int32),
            pltpu.VMEM((2, _CH, _D // 2), jnp.int32),
            pltpu.SemaphoreType.DMA,
        ],
    )
    return f(x2d, p1r, p2r)


_FB = 1024


def _cast_kernel(w1_ref, w2_ref, o1_ref, o2_ref):
    o1_ref[...] = w1_ref[...].astype(jnp.bfloat16)
    o2_ref[...] = w2_ref[...].astype(jnp.bfloat16)


def _run_cast(W1, W2):
    return pl.pallas_call(
        _cast_kernel,
        grid=(_E, _F // _FB),
        in_specs=[
            pl.BlockSpec((1, _D, _FB), lambda e, j: (e, 0, j)),
            pl.BlockSpec((1, _FB, _D), lambda e, j: (e, j, 0)),
        ],
        out_specs=[
            pl.BlockSpec((1, _D, _FB), lambda e, j: (e, 0, j)),
            pl.BlockSpec((1, _FB, _D), lambda e, j: (e, j, 0)),
        ],
        out_shape=[
            jax.ShapeDtypeStruct((_E, _D, _F), jnp.bfloat16),
            jax.ShapeDtypeStruct((_E, _F, _D), jnp.bfloat16),
        ],
    )(W1, W2)


def _gffn_kernel(te_ref, xs_ref, w1_ref, b1_ref, w2_ref, b2_ref,
                 out_ref):
    x = xs_ref[...]
    acc = jnp.zeros((_TM2, _D), jnp.float32)
    fj = 1024
    for j in range(_F // fj):
        h = lax.dot_general(x, w1_ref[0, :, j * fj:(j + 1) * fj],
                            (((1,), (0,)), ((), ())),
                            preferred_element_type=jnp.float32)
        h = jax.nn.gelu(h + b1_ref[0, :, j * fj:(j + 1) * fj])
        acc = acc + lax.dot_general(h.astype(jnp.bfloat16),
                                    w2_ref[0, j * fj:(j + 1) * fj, :],
                                    (((1,), (0,)), ((), ())),
                                    preferred_element_type=jnp.float32)
    out_ref[...] = acc + b2_ref[0]


def _run_gffn(te, xs, w1b, b1r, w2b, b2r):
    grid_spec = pltpu.PrefetchScalarGridSpec(
        num_scalar_prefetch=1,
        grid=(_NTILE,),
        in_specs=[
            pl.BlockSpec((_TM2, _D), lambda i, te_r: (i, 0)),
            pl.BlockSpec((1, _D, _F), lambda i, te_r: (te_r[i], 0, 0)),
            pl.BlockSpec((1, 1, _F), lambda i, te_r: (te_r[i], 0, 0)),
            pl.BlockSpec((1, _F, _D), lambda i, te_r: (te_r[i], 0, 0)),
            pl.BlockSpec((1, 1, _D), lambda i, te_r: (te_r[i], 0, 0)),
        ],
        out_specs=pl.BlockSpec((_TM2, _D), lambda i, te_r: (i, 0)),
    )
    return pl.pallas_call(
        _gffn_kernel,
        grid_spec=grid_spec,
        out_shape=jax.ShapeDtypeStruct((_NSLOT, _D), jnp.float32),
    )(te, xs, w1b, b1r, w2b, b2r)


def _combine_kernel(ys_hbm, p1_hbm, p2_hbm, w1_hbm, w2_hbm, out_hbm,
                    pb1, pb2, wb1, wb2, buf1, buf2, sem1, sem2, semo):
    wid = lax.axis_index("s") * _NC + lax.axis_index("c")
    tb = wid * _TPW
    pltpu.sync_copy(p1_hbm.at[pl.ds(wid * _NCH2, _NCH2)], pb1)
    pltpu.sync_copy(p2_hbm.at[pl.ds(wid * _NCH2, _NCH2)], pb2)
    pltpu.sync_copy(w1_hbm.at[pl.ds(tb, _TPW)], wb1)
    pltpu.sync_copy(w2_hbm.at[pl.ds(tb, _TPW)], wb2)
    cp1 = {0: pltpu.async_copy(ys_hbm.at[pb1.at[0]], buf1.at[0], sem1)}
    cp2 = {0: pltpu.async_copy(ys_hbm.at[pb2.at[0]], buf2.at[0], sem2)}
    cpo = {}
    for c in range(_NCH2):
        b = c % 2
        cp1[c].wait()
        cp2[c].wait()
        if c + 1 < _NCH2:
            nb = (c + 1) % 2
            if c >= 1:
                cpo[c - 1].wait()
            cp1[c + 1] = pltpu.async_copy(ys_hbm.at[pb1.at[c + 1]],
                                          buf1.at[nb], sem1)
            cp2[c + 1] = pltpu.async_copy(ys_hbm.at[pb2.at[c + 1]],
                                          buf2.at[nb], sem2)

        def row_body(r, _):
            wv1 = wb1[c * _CH2 + r, pl.ds(0, 16)]
            wv2 = wb2[c * _CH2 + r, pl.ds(0, 16)]

            def lane_body(l, _):
                buf1[b, r, pl.ds(l * 16, 16)] = (
                    wv1 * buf1[b, r, pl.ds(l * 16, 16)]
                    + wv2 * buf2[b, r, pl.ds(l * 16, 16)])
                return 0

            lax.fori_loop(0, _D // 16, lane_body, 0, unroll=8)
            return 0

        lax.fori_loop(0, _CH2, row_body, 0)
        cpo[c] = pltpu.async_copy(buf1.at[b],
                                  out_hbm.at[pl.ds(tb + _CH2 * c, _CH2)],
                                  semo)
    cpo[_NCH2 - 2].wait()
    cpo[_NCH2 - 1].wait()


def _run_combine(ys, p1r, p2r, w1rep, w2rep):
    mesh = plsc.VectorSubcoreMesh(core_axis_name="c", subcore_axis_name="s")
    f = pl.kernel(
        _combine_kernel,
        mesh=mesh,
        out_type=jax.ShapeDtypeStruct((_N, _D), jnp.float32),
        scratch_types=[
            pltpu.VMEM((_NCH2, _CH2), jnp.int32),
            pltpu.VMEM((_NCH2, _CH2), jnp.int32),
            pltpu.VMEM((_TPW, _EP), jnp.float32),
            pltpu.VMEM((_TPW, _EP), jnp.float32),
            pltpu.VMEM((2, _CH2, _D), jnp.float32),
            pltpu.VMEM((2, _CH2, _D), jnp.float32),
            pltpu.SemaphoreType.DMA,
            pltpu.SemaphoreType.DMA,
            pltpu.SemaphoreType.DMA,
        ],
    )
    return f(ys, p1r, p2r, w1rep, w2rep)


def kernel(inputs, task_param, Wg_txt, bg_txt, Wg_img, bg_img, Wt, bt,
           alpha, W1, b1, W2, b2, is_text):
    x2d = inputs.reshape(_N, _D)
    padE = ((0, 0), (0, _EP - _E))
    wgt = jnp.pad(Wg_txt, padE)
    wgi = jnp.pad(Wg_img, padE)
    wt = jnp.pad(Wt, padE)
    bgt = jnp.pad(bg_txt, (0, _EP - _E)).reshape(1, _EP)
    bgi = jnp.pad(bg_img, (0, _EP - _E)).reshape(1, _EP)
    btp = jnp.pad(bt, (0, _EP - _E)).reshape(1, _EP)
    tp = task_param.reshape(1, _D)
    a = jnp.asarray(alpha, jnp.float32).reshape(1, 1)
    it = jnp.asarray(is_text, jnp.int32).reshape(1, 1)

    mi, mf, _stats, _cntc, laux, basel, te = _run_gate(
        x2d, wgt, wgi, bgt, bgi, tp, wt, btp, a, it)

    p1, p2, w1rep, w2rep, x2b = _run_pos(mi, mf, basel, x2d)
    p1r = p1.reshape(_N // _CH, _CH)
    p2r = p2.reshape(_N // _CH, _CH)
    p1c = p1.reshape(_N // _CH2, _CH2)
    p2c = p2.reshape(_N // _CH2, _CH2)
    te_arr = te[0, :_NTILE]

    w1b, w2b = _run_cast(W1, W2)

    xi = lax.bitcast_convert_type(x2b.reshape(_N, _D // 2, 2), jnp.int32)

    xsi = _run_dispatch(xi, p1r, p2r)
    xs = lax.bitcast_convert_type(xsi, jnp.bfloat16).reshape(_NSLOT, _D)

    ys = _run_gffn(te_arr, xs, w1b,
                   b1.reshape(_E, 1, _F), w2b,
                   b2.reshape(_E, 1, _D))

    out2d = _run_combine(ys, p1c, p2c, w1rep, w2rep)

    return out2d.reshape(_L, _B, _D), laux[0, 0]


# speedup vs baseline: 1.5917x; 1.5917x over previous
import jax
import jax.numpy as jnp
from jax import lax
from jax.experimental import pallas as pl
from jax.experimental.pallas import tpu as pltpu
from jax.experimental.pallas import tpu_sc as plsc

_L, _B, _D = 2048, 2, 1024
_E, _K, _F = 8, 2, 4096
_N = _L * _B
_EP = 128
_TM = 512
_NT = _N // _TM
_NEG = -1e30

_TM2 = 256
_NTILE = _N * _K // _TM2 + _E
_NSLOT = _NTILE * _TM2

_NC, _NSUB = 2, 16
_NW = _NC * _NSUB
_TPW = _N // _NW
_CH = 32
_NCH = _TPW // _CH
_CH2 = 16
_NCH2 = _TPW // _CH2


def _gate_kernel(x_ref, wgt_ref, wgi_ref, bgt_ref, bgi_ref, tp_ref, wt_ref,
                 bt_ref, alpha_ref, it_ref,
                 mi_ref, mf_ref, stats_ref, cntc_ref, laux_ref, basel_ref,
                 te_ref):
    t = pl.program_id(0)
    a = alpha_ref[0, 0]
    use_txt = it_ref[0, 0] != 0
    wg = jnp.where(use_txt, wgt_ref[...], wgi_ref[...])
    bg = jnp.where(use_txt, bgt_ref[...], bgi_ref[...])
    x = x_ref[...]
    il = lax.dot_general(x, wg, (((1,), (0,)), ((), ())),
                         preferred_element_type=jnp.float32) + bg
    tl = lax.dot_general(tp_ref[...], wt_ref[...], (((1,), (0,)), ((), ())),
                         preferred_element_type=jnp.float32) + bt_ref[...]
    g = (1.0 - a) * il + a * tl
    lane = lax.broadcasted_iota(jnp.int32, (_TM, _EP), 1)
    valid = lane < _E
    g = jnp.where(valid, g, _NEG)
    m1 = jnp.max(g, axis=1, keepdims=True)
    i1 = jnp.min(jnp.where(g == m1, lane, _EP), axis=1, keepdims=True)
    sel1 = lane == i1
    g2 = jnp.where(sel1, _NEG, g)
    m2 = jnp.max(g2, axis=1, keepdims=True)
    i2 = jnp.min(jnp.where(g2 == m2, lane, _EP), axis=1, keepdims=True)
    sel2 = lane == i2
    d = jnp.exp(m2 - m1)
    w1 = 1.0 / (1.0 + d)
    w2 = d / (1.0 + d)
    ez = jnp.where(valid, jnp.exp(g - m1), 0.0)
    p = ez / jnp.sum(ez, axis=1, keepdims=True)
    sump = jnp.sum(p, axis=0, keepdims=True)
    m = jnp.where(sel1 | sel2, 1.0, 0.0)
    cnt = jnp.sum(m, axis=0, keepdims=True)

    @pl.when(t == 0)
    def _():
        stats_ref[...] = jnp.zeros_like(stats_ref)
        cntc_ref[...] = jnp.zeros_like(cntc_ref)

    @pl.when(t < _NT)
    def _():
        carry = stats_ref[1:2, :]
        row = lax.broadcasted_iota(jnp.int32, (_TM, _TM), 0)
        col = lax.broadcasted_iota(jnp.int32, (_TM, _TM), 1)
        ltri = jnp.where(col < row, 1.0, 0.0).astype(jnp.bfloat16)
        excl = lax.dot_general(ltri, m.astype(jnp.bfloat16),
                               (((1,), (0,)), ((), ())),
                               preferred_element_type=jnp.float32)
        rank = carry + excl
        r1 = jnp.sum(jnp.where(sel1, rank, 0.0), axis=1, keepdims=True)
        r2 = jnp.sum(jnp.where(sel2, rank, 0.0), axis=1, keepdims=True)
        lane8 = lax.broadcasted_iota(jnp.int32, (_TM, 8), 1)
        mi = (jnp.where(lane8 == 0, i1, 0) + jnp.where(lane8 == 1, i2, 0)
              + jnp.where(lane8 == 2, r1.astype(jnp.int32), 0)
              + jnp.where(lane8 == 3, r2.astype(jnp.int32), 0))
        mf = jnp.where(lane8 == 0, w1, 0.0) + jnp.where(lane8 == 1, w2, 0.0)
        mi_ref[...] = mi
        mf_ref[...] = mf
        stats_ref[...] += jnp.concatenate([sump, cnt], axis=0)
        cntc_ref[...] += lax.dot_general(
            m.astype(jnp.bfloat16), jnp.ones((_TM, _EP), jnp.bfloat16),
            (((0,), (0,)), ((), ())), preferred_element_type=jnp.float32)

    @pl.when(t == _NT)
    def _():
        s = stats_ref[...]
        lx = jnp.sum(s[0:1, :] * s[1:2, :]) * (1.0 / (_N * _N))
        laux_ref[...] = jnp.full((1, _EP), lx, jnp.float32)
        sub = lax.broadcasted_iota(jnp.int32, (_EP, _EP), 0)
        lan = lax.broadcasted_iota(jnp.int32, (_EP, _EP), 1)
        cc = jnp.where(sub < _E, cntc_ref[...], 0.0)
        cpad = jnp.ceil(cc * (1.0 / _TM2)) * float(_TM2)
        ustrict = jnp.where(sub < lan, 1.0, 0.0).astype(jnp.bfloat16)
        bases = lax.dot_general(ustrict, cpad.astype(jnp.bfloat16),
                                (((0,), (0,)), ((), ())),
                                preferred_element_type=jnp.float32)
        basel = jnp.sum(jnp.where(sub == lan, bases, 0.0), axis=0,
                        keepdims=True)
        basel_ref[...] = basel.astype(jnp.int32)
        btiles = bases * (1.0 / _TM2)
        cmp = jnp.where((sub < _E) & (btiles <= lan.astype(jnp.float32)),
                        1.0, 0.0)
        te_ref[...] = (jnp.sum(cmp, axis=0, keepdims=True)
                       - 1.0).astype(jnp.int32)


def _run_gate(x2d, wgt, wgi, bgt, bgi, tp, wt, btp, a, it):
    last = _NT - 1
    return pl.pallas_call(
        _gate_kernel,
        grid=(_NT + 1,),
        in_specs=[
            pl.BlockSpec((_TM, _D), lambda t: (jnp.minimum(t, last), 0)),
            pl.BlockSpec((_D, _EP), lambda t: (0, 0)),
            pl.BlockSpec((_D, _EP), lambda t: (0, 0)),
            pl.BlockSpec((1, _EP), lambda t: (0, 0)),
            pl.BlockSpec((1, _EP), lambda t: (0, 0)),
            pl.BlockSpec((1, _D), lambda t: (0, 0)),
            pl.BlockSpec((_D, _EP), lambda t: (0, 0)),
            pl.BlockSpec((1, _EP), lambda t: (0, 0)),
            pl.BlockSpec((1, 1), lambda t: (0, 0)),
            pl.BlockSpec((1, 1), lambda t: (0, 0)),
        ],
        out_specs=[
            pl.BlockSpec((_TM, 8), lambda t: (jnp.minimum(t, last), 0)),
            pl.BlockSpec((_TM, 8), lambda t: (jnp.minimum(t, last), 0)),
            pl.BlockSpec((2, _EP), lambda t: (0, 0)),
            pl.BlockSpec((_EP, _EP), lambda t: (0, 0)),
            pl.BlockSpec((1, _EP), lambda t: (0, 0)),
            pl.BlockSpec((1, _EP), lambda t: (0, 0)),
            pl.BlockSpec((1, _EP), lambda t: (0, 0)),
        ],
        out_shape=[
            jax.ShapeDtypeStruct((_N, 8), jnp.int32),
            jax.ShapeDtypeStruct((_N, 8), jnp.float32),
            jax.ShapeDtypeStruct((2, _EP), jnp.float32),
            jax.ShapeDtypeStruct((_EP, _EP), jnp.float32),
            jax.ShapeDtypeStruct((1, _EP), jnp.float32),
            jax.ShapeDtypeStruct((1, _EP), jnp.int32),
            jax.ShapeDtypeStruct((1, _EP), jnp.int32),
        ],
    )(x2d, wgt, wgi, bgt, bgi, tp, wt, btp, a, it)


def _pos_kernel(mi_ref, mf_ref, basel_ref, p1_ref, p2_ref, w1_ref, w2_ref):
    mi = mi_ref[...]
    mf = mf_ref[...]
    lane8 = lax.broadcasted_iota(jnp.int32, (_TM, 8), 1)
    e1 = jnp.sum(jnp.where(lane8 == 0, mi, 0), axis=1, keepdims=True)
    e2 = jnp.sum(jnp.where(lane8 == 1, mi, 0), axis=1, keepdims=True)
    r1 = jnp.sum(jnp.where(lane8 == 2, mi, 0), axis=1, keepdims=True)
    r2 = jnp.sum(jnp.where(lane8 == 3, mi, 0), axis=1, keepdims=True)
    wv1 = jnp.sum(jnp.where(lane8 == 0, mf, 0.0), axis=1, keepdims=True)
    wv2 = jnp.sum(jnp.where(lane8 == 1, mf, 0.0), axis=1, keepdims=True)
    lane = lax.broadcasted_iota(jnp.int32, (_TM, _EP), 1)
    b = basel_ref[...]
    b1 = jnp.sum(jnp.where(lane == e1, b, 0), axis=1, keepdims=True)
    b2 = jnp.sum(jnp.where(lane == e2, b, 0), axis=1, keepdims=True)
    p1_ref[...] = b1 + r1
    p2_ref[...] = b2 + r2
    w1_ref[...] = jnp.broadcast_to(wv1, (_TM, _EP))
    w2_ref[...] = jnp.broadcast_to(wv2, (_TM, _EP))


def _run_pos(mi, mf, basel):
    return pl.pallas_call(
        _pos_kernel,
        grid=(_NT,),
        in_specs=[
            pl.BlockSpec((_TM, 8), lambda t: (t, 0)),
            pl.BlockSpec((_TM, 8), lambda t: (t, 0)),
            pl.BlockSpec((1, _EP), lambda t: (0, 0)),
        ],
        out_specs=[
            pl.BlockSpec((_TM, 1), lambda t: (t, 0)),
            pl.BlockSpec((_TM, 1), lambda t: (t, 0)),
            pl.BlockSpec((_TM, _EP), lambda t: (t, 0)),
            pl.BlockSpec((_TM, _EP), lambda t: (t, 0)),
        ],
        out_shape=[
            jax.ShapeDtypeStruct((_N, 1), jnp.int32),
            jax.ShapeDtypeStruct((_N, 1), jnp.int32),
            jax.ShapeDtypeStruct((_N, _EP), jnp.float32),
            jax.ShapeDtypeStruct((_N, _EP), jnp.float32),
        ],
    )(mi, mf, basel)


def _dispatch_kernel(x_hbm, p1_hbm, p2_hbm, w1_hbm, w2_hbm,
                     xs_hbm, ws_hbm,
                     pb1, pb2, xbuf, wbuf1, wbuf2, sem):
    wid = lax.axis_index("s") * _NC + lax.axis_index("c")
    tb = wid * _TPW
    pltpu.sync_copy(p1_hbm.at[pl.ds(wid * _NCH, _NCH)], pb1)
    pltpu.sync_copy(p2_hbm.at[pl.ds(wid * _NCH, _NCH)], pb2)
    cps = []
    for c in range(_NCH):
        b = c % 2
        if c >= 2:
            for cp in cps[4 * (c - 2):4 * (c - 1)]:
                cp.wait()
        pltpu.sync_copy(x_hbm.at[pl.ds(tb + _CH * c, _CH)], xbuf.at[b])
        pltpu.sync_copy(w1_hbm.at[pl.ds(tb + _CH * c, _CH)], wbuf1.at[b])
        pltpu.sync_copy(w2_hbm.at[pl.ds(tb + _CH * c, _CH)], wbuf2.at[b])
        cps.append(pltpu.async_copy(xbuf.at[b], xs_hbm.at[pb1.at[c]], sem))
        cps.append(pltpu.async_copy(xbuf.at[b], xs_hbm.at[pb2.at[c]], sem))
        cps.append(pltpu.async_copy(wbuf1.at[b], ws_hbm.at[pb1.at[c]], sem))
        cps.append(pltpu.async_copy(wbuf2.at[b], ws_hbm.at[pb2.at[c]], sem))
    for cp in cps[4 * (_NCH - 2):]:
        cp.wait()


def _run_dispatch(x2d, p1r, p2r, w1rep, w2rep):
    mesh = plsc.VectorSubcoreMesh(core_axis_name="c", subcore_axis_name="s")
    f = pl.kernel(
        _dispatch_kernel,
        mesh=mesh,
        out_type=[
            jax.ShapeDtypeStruct((_NSLOT, _D), jnp.float32),
            jax.ShapeDtypeStruct((_NSLOT, _EP), jnp.float32),
        ],
        scratch_types=[
            pltpu.VMEM((_NCH, _CH), jnp.int32),
            pltpu.VMEM((_NCH, _CH), jnp.int32),
            pltpu.VMEM((2, _CH, _D), jnp.float32),
            pltpu.VMEM((2, _CH, _EP), jnp.float32),
            pltpu.VMEM((2, _CH, _EP), jnp.float32),
            pltpu.SemaphoreType.DMA,
        ],
    )
    return f(x2d, p1r, p2r, w1rep, w2rep)


_FB = 1024


def _cast_kernel(w1_ref, w2_ref, o1_ref, o2_ref):
    o1_ref[...] = w1_ref[...].astype(jnp.bfloat16)
    o2_ref[...] = w2_ref[...].astype(jnp.bfloat16)


def _run_cast(W1, W2):
    return pl.pallas_call(
        _cast_kernel,
        grid=(_E, _F // _FB),
        in_specs=[
            pl.BlockSpec((1, _D, _FB), lambda e, j: (e, 0, j)),
            pl.BlockSpec((1, _FB, _D), lambda e, j: (e, j, 0)),
        ],
        out_specs=[
            pl.BlockSpec((1, _D, _FB), lambda e, j: (e, 0, j)),
            pl.BlockSpec((1, _FB, _D), lambda e, j: (e, j, 0)),
        ],
        out_shape=[
            jax.ShapeDtypeStruct((_E, _D, _F), jnp.bfloat16),
            jax.ShapeDtypeStruct((_E, _F, _D), jnp.bfloat16),
        ],
    )(W1, W2)


def _gffn_kernel(te_ref, xs_ref, ws_ref, w1_ref, b1_ref, w2_ref, b2_ref,
                 out_ref):
    x = xs_ref[...].astype(jnp.bfloat16)
    acc = jnp.zeros((_TM2, _D), jnp.float32)
    fj = 1024
    for j in range(_F // fj):
        h = lax.dot_general(x, w1_ref[0, :, j * fj:(j + 1) * fj],
                            (((1,), (0,)), ((), ())),
                            preferred_element_type=jnp.float32)
        h = jax.nn.gelu(h + b1_ref[0, :, j * fj:(j + 1) * fj])
        acc = acc + lax.dot_general(h.astype(jnp.bfloat16),
                                    w2_ref[0, j * fj:(j + 1) * fj, :],
                                    (((1,), (0,)), ((), ())),
                                    preferred_element_type=jnp.float32)
    out_ref[...] = (acc + b2_ref[0]) * ws_ref[:, 0:1]


def _run_gffn(te, xs, ws, w1b, b1r, w2b, b2r):
    grid_spec = pltpu.PrefetchScalarGridSpec(
        num_scalar_prefetch=1,
        grid=(_NTILE,),
        in_specs=[
            pl.BlockSpec((_TM2, _D), lambda i, te_r: (i, 0)),
            pl.BlockSpec((_TM2, _EP), lambda i, te_r: (i, 0)),
            pl.BlockSpec((1, _D, _F), lambda i, te_r: (te_r[i], 0, 0)),
            pl.BlockSpec((1, 1, _F), lambda i, te_r: (te_r[i], 0, 0)),
            pl.BlockSpec((1, _F, _D), lambda i, te_r: (te_r[i], 0, 0)),
            pl.BlockSpec((1, 1, _D), lambda i, te_r: (te_r[i], 0, 0)),
        ],
        out_specs=pl.BlockSpec((_TM2, _D), lambda i, te_r: (i, 0)),
    )
    return pl.pallas_call(
        _gffn_kernel,
        grid_spec=grid_spec,
        out_shape=jax.ShapeDtypeStruct((_NSLOT, _D), jnp.float32),
    )(te, xs, ws, w1b, b1r, w2b, b2r)


def _combine_kernel(ys_hbm, p1_hbm, p2_hbm, out_hbm,
                    pb1, pb2, buf1, buf2, sem1, sem2, semo):
    wid = lax.axis_index("s") * _NC + lax.axis_index("c")
    tb = wid * _TPW
    pltpu.sync_copy(p1_hbm.at[pl.ds(wid * _NCH2, _NCH2)], pb1)
    pltpu.sync_copy(p2_hbm.at[pl.ds(wid * _NCH2, _NCH2)], pb2)
    cp1 = {0: pltpu.async_copy(ys_hbm.at[pb1.at[0]], buf1.at[0], sem1)}
    cp2 = {0: pltpu.async_copy(ys_hbm.at[pb2.at[0]], buf2.at[0], sem2)}
    cpo = {}
    for c in range(_NCH2):
        b = c % 2
        cp1[c].wait()
        cp2[c].wait()
        if c + 1 < _NCH2:
            nb = (c + 1) % 2
            if c >= 1:
                cpo[c - 1].wait()
            cp1[c + 1] = pltpu.async_copy(ys_hbm.at[pb1.at[c + 1]],
                                          buf1.at[nb], sem1)
            cp2[c + 1] = pltpu.async_copy(ys_hbm.at[pb2.at[c + 1]],
                                          buf2.at[nb], sem2)

        def row_body(r, _):
            def lane_body(l, _):
                buf1[b, r, pl.ds(l * 16, 16)] = (
                    buf1[b, r, pl.ds(l * 16, 16)]
                    + buf2[b, r, pl.ds(l * 16, 16)])
                return 0

            lax.fori_loop(0, _D // 16, lane_body, 0, unroll=8)
            return 0

        lax.fori_loop(0, _CH2, row_body, 0)
        cpo[c] = pltpu.async_copy(buf1.at[b],
                                  out_hbm.at[pl.ds(tb + _CH2 * c, _CH2)],
                                  semo)
    cpo[_NCH2 - 2].wait()
    cpo[_NCH2 - 1].wait()


def _run_combine(ys, p1r, p2r):
    mesh = plsc.VectorSubcoreMesh(core_axis_name="c", subcore_axis_name="s")
    f = pl.kernel(
        _combine_kernel,
        mesh=mesh,
        out_type=jax.ShapeDtypeStruct((_N, _D), jnp.float32),
        scratch_types=[
            pltpu.VMEM((_NCH2, _CH2), jnp.int32),
            pltpu.VMEM((_NCH2, _CH2), jnp.int32),
            pltpu.VMEM((2, _CH2, _D), jnp.float32),
            pltpu.VMEM((2, _CH2, _D), jnp.float32),
            pltpu.SemaphoreType.DMA,
            pltpu.SemaphoreType.DMA,
            pltpu.SemaphoreType.DMA,
        ],
    )
    return f(ys, p1r, p2r)


def kernel(inputs, task_param, Wg_txt, bg_txt, Wg_img, bg_img, Wt, bt,
           alpha, W1, b1, W2, b2, is_text):
    x2d = inputs.reshape(_N, _D)
    padE = ((0, 0), (0, _EP - _E))
    wgt = jnp.pad(Wg_txt, padE)
    wgi = jnp.pad(Wg_img, padE)
    wt = jnp.pad(Wt, padE)
    bgt = jnp.pad(bg_txt, (0, _EP - _E)).reshape(1, _EP)
    bgi = jnp.pad(bg_img, (0, _EP - _E)).reshape(1, _EP)
    btp = jnp.pad(bt, (0, _EP - _E)).reshape(1, _EP)
    tp = task_param.reshape(1, _D)
    a = jnp.asarray(alpha, jnp.float32).reshape(1, 1)
    it = jnp.asarray(is_text, jnp.int32).reshape(1, 1)

    mi, mf, _stats, _cntc, laux, basel, te = _run_gate(
        x2d, wgt, wgi, bgt, bgi, tp, wt, btp, a, it)

    p1, p2, w1rep, w2rep = _run_pos(mi, mf, basel)
    p1r = p1.reshape(_N // _CH, _CH)
    p2r = p2.reshape(_N // _CH, _CH)
    p1c = p1.reshape(_N // _CH2, _CH2)
    p2c = p2.reshape(_N // _CH2, _CH2)
    te_arr = te[0, :_NTILE]

    w1b, w2b = _run_cast(W1, W2)

    xs, ws = _run_dispatch(x2d, p1r, p2r, w1rep, w2rep)

    ys = _run_gffn(te_arr, xs, ws, w1b,
                   b1.reshape(_E, 1, _F), w2b,
                   b2.reshape(_E, 1, _D))

    out2d = _run_combine(ys, p1c, p2c)

    return out2d.reshape(_L, _B, _D), laux[0, 0]
